# baseline (speedup 1.0000x reference)
import functools

import numpy as np
import jax
import jax.numpy as jnp
from jax import lax
from jax.experimental import pallas as pl
from jax.experimental.pallas import tpu as pltpu
from jax.experimental.pallas import tpu_sc as plsc

_BATCH = 32
_CH = 3
_NPOINT = 65536
_ROWS = _BATCH * _CH
_NTILES = 32
_ROWS_PER_TILE = _ROWS // _NTILES
_LANES = 16


@functools.lru_cache(maxsize=None)
def _make_idx() -> np.ndarray:
    try:
        cpu = jax.devices("cpu")[0]
        ctx = jax.default_device(cpu)
    except Exception:
        import contextlib
        ctx = contextlib.nullcontext()
    with ctx:
        key = jax.random.key(42)
        k_theta, k_perm = jax.random.split(key)
        theta = float(jax.random.uniform(k_theta, (), minval=0.0, maxval=0.95))
        new_npoint = int((1.0 - theta) * _NPOINT)
        perm = jax.random.permutation(k_perm, _NPOINT)
        return np.asarray(perm[:new_npoint], dtype=np.int32)


_IDX_NP = _make_idx()


@functools.lru_cache(maxsize=None)
def _build():
    idx_np = _IDX_NP
    m = int(idx_np.shape[0])
    mp = ((m + _LANES - 1) // _LANES) * _LANES
    idx_pad = np.zeros((mp,), np.int32)
    idx_pad[:m] = idx_np

    mesh = plsc.VectorSubcoreMesh(core_axis_name="c", subcore_axis_name="s")

    @functools.partial(
        pl.kernel,
        out_type=jax.ShapeDtypeStruct((_ROWS, m), jnp.float32),
        mesh=mesh,
        compiler_params=pltpu.CompilerParams(
            needs_layout_passes=False, use_tc_tiling_on_sc=False),
        scratch_types=[
            pltpu.VMEM((_NPOINT,), jnp.float32),
            pltpu.VMEM((mp,), jnp.int32),
            pltpu.VMEM((mp,), jnp.float32),
        ],
    )
    def _points_gather(x_hbm, idx_hbm, out_hbm, row_v, idx_v, out_v):
        wid = lax.axis_index("s") * 2 + lax.axis_index("c")
        pltpu.sync_copy(idx_hbm, idx_v)

        for r in range(_ROWS_PER_TILE):
            row = wid * _ROWS_PER_TILE + r
            pltpu.sync_copy(x_hbm.at[row], row_v)

            def body(j, carry):
                iv = idx_v[pl.ds(j * _LANES, _LANES)]
                out_v[pl.ds(j * _LANES, _LANES)] = plsc.load_gather(row_v, [iv])
                return carry

            lax.fori_loop(0, mp // _LANES, body, 0, unroll=8)
            pltpu.sync_copy(out_v.at[pl.ds(0, m)], out_hbm.at[row])

    return _points_gather, idx_pad, m


def kernel(xyz):
    points_gather, idx_pad, m = _build()
    x2 = xyz.reshape(_ROWS, _NPOINT)
    idx = jnp.asarray(idx_pad)
    out = points_gather(x2, idx)
    return out.reshape(_BATCH, _CH, m)


# speedup vs baseline: 1.0973x; 1.0973x over previous
import functools

import numpy as np
import jax
import jax.numpy as jnp
from jax import lax
from jax.experimental import pallas as pl
from jax.experimental.pallas import tpu as pltpu
from jax.experimental.pallas import tpu_sc as plsc

_BATCH = 32
_CH = 3
_NPOINT = 65536
_ROWS = _BATCH * _CH
_NTILES = 32
_LANES = 16


@functools.lru_cache(maxsize=None)
def _make_idx() -> np.ndarray:
    try:
        cpu = jax.devices("cpu")[0]
        ctx = jax.default_device(cpu)
    except Exception:
        import contextlib
        ctx = contextlib.nullcontext()
    with ctx:
        key = jax.random.key(42)
        k_theta, k_perm = jax.random.split(key)
        theta = float(jax.random.uniform(k_theta, (), minval=0.0, maxval=0.95))
        new_npoint = int((1.0 - theta) * _NPOINT)
        perm = jax.random.permutation(k_perm, _NPOINT)
        return np.asarray(perm[:new_npoint], dtype=np.int32)


_IDX_NP = _make_idx()


@functools.lru_cache(maxsize=None)
def _build():
    idx_np = _IDX_NP
    m = int(idx_np.shape[0])
    mp = ((m + 127) // 128) * 128
    idx_pad = np.zeros((mp,), np.int32)
    idx_pad[:m] = idx_np

    mesh = plsc.VectorSubcoreMesh(core_axis_name="c", subcore_axis_name="s")

    @functools.partial(
        pl.kernel,
        out_type=jax.ShapeDtypeStruct((_ROWS, mp), jnp.float32),
        mesh=mesh,
        compiler_params=pltpu.CompilerParams(needs_layout_passes=False),
        scratch_types=[
            pltpu.VMEM((_NPOINT,), jnp.float32),
            pltpu.VMEM((mp,), jnp.int32),
            pltpu.VMEM((mp,), jnp.float32),
        ],
    )
    def _points_gather(x_hbm, idx_hbm, out_hbm, row_v, idx_v, out_v):
        wid = lax.axis_index("s") * 2 + lax.axis_index("c")
        pltpu.sync_copy(idx_hbm, idx_v)

        for r in range(_CH):
            row = wid * _CH + r
            pltpu.sync_copy(x_hbm.at[row], row_v)

            def body(j, carry):
                iv = idx_v[pl.ds(j * _LANES, _LANES)]
                out_v[pl.ds(j * _LANES, _LANES)] = plsc.load_gather(row_v, [iv])
                return carry

            lax.fori_loop(0, mp // _LANES, body, 0, unroll=8)
            pltpu.sync_copy(out_v, out_hbm.at[row])

    return _points_gather, idx_pad, m


def kernel(xyz):
    points_gather, idx_pad, m = _build()
    x2 = xyz.reshape(_ROWS, _NPOINT)
    idx = jnp.asarray(idx_pad)
    out = points_gather(x2, idx)
    return out[:, :m].reshape(_BATCH, _CH, m)


# speedup vs baseline: 1.4234x; 1.2973x over previous
import functools

import numpy as np
import jax
import jax.numpy as jnp
from jax import lax
from jax.experimental import pallas as pl
from jax.experimental.pallas import tpu as pltpu
from jax.experimental.pallas import tpu_sc as plsc

_BATCH = 32
_CH = 3
_NPOINT = 65536
_ROWS = _BATCH * _CH
_LANES = 16


@functools.lru_cache(maxsize=None)
def _make_idx() -> np.ndarray:
    try:
        cpu = jax.devices("cpu")[0]
        ctx = jax.default_device(cpu)
    except Exception:
        import contextlib
        ctx = contextlib.nullcontext()
    with ctx:
        key = jax.random.key(42)
        k_theta, k_perm = jax.random.split(key)
        theta = float(jax.random.uniform(k_theta, (), minval=0.0, maxval=0.95))
        new_npoint = int((1.0 - theta) * _NPOINT)
        perm = jax.random.permutation(k_perm, _NPOINT)
        return np.asarray(perm[:new_npoint], dtype=np.int32)


_IDX_NP = _make_idx()


@functools.lru_cache(maxsize=None)
def _build():
    idx_np = _IDX_NP
    m = int(idx_np.shape[0])
    mp = ((m + 127) // 128) * 128
    idx_pad = np.zeros((mp,), np.int32)
    idx_pad[:m] = idx_np

    mesh = plsc.VectorSubcoreMesh(core_axis_name="c", subcore_axis_name="s")

    @functools.partial(
        pl.kernel,
        out_type=jax.ShapeDtypeStruct((_ROWS, mp), jnp.float32),
        mesh=mesh,
        compiler_params=pltpu.CompilerParams(needs_layout_passes=False),
        scratch_types=[
            pltpu.VMEM((_NPOINT,), jnp.float32),
            pltpu.VMEM((mp,), jnp.int32),
            pltpu.VMEM((mp,), jnp.float32),
        ],
    )
    def _points_gather(x_hbm, idx_hbm, out_hbm, row_v, idx_v, out_v):
        wid = lax.axis_index("s") * 2 + lax.axis_index("c")
        pltpu.sync_copy(idx_hbm, idx_v)

        for r in range(_CH):
            row = wid * _CH + r
            pltpu.sync_copy(x_hbm.at[row], row_v)

            @plsc.parallel_loop(0, mp, step=_LANES, unroll=8)
            def _gather_step(i):
                iv = idx_v[pl.ds(i, _LANES)]
                out_v[pl.ds(i, _LANES)] = plsc.load_gather(row_v, [iv])

            pltpu.sync_copy(out_v, out_hbm.at[row])

    return _points_gather, idx_pad, m


def kernel(xyz):
    points_gather, idx_pad, m = _build()
    x2 = xyz.reshape(_ROWS, _NPOINT)
    idx = jnp.asarray(idx_pad)
    out = points_gather(x2, idx)
    return out[:, :m].reshape(_BATCH, _CH, m)


# speedup vs baseline: 1.6066x; 1.1287x over previous
import functools

import numpy as np
import jax
import jax.numpy as jnp
from jax import lax
from jax.experimental import pallas as pl
from jax.experimental.pallas import tpu as pltpu
from jax.experimental.pallas import tpu_sc as plsc

_BATCH = 32
_CH = 3
_NPOINT = 65536
_ROWS = _BATCH * _CH
_LANES = 16
_W = 4096
_NW = _NPOINT // _W


@functools.lru_cache(maxsize=None)
def _make_idx() -> np.ndarray:
    try:
        cpu = jax.devices("cpu")[0]
        ctx = jax.default_device(cpu)
    except Exception:
        import contextlib
        ctx = contextlib.nullcontext()
    with ctx:
        key = jax.random.key(42)
        k_theta, k_perm = jax.random.split(key)
        theta = float(jax.random.uniform(k_theta, (), minval=0.0, maxval=0.95))
        new_npoint = int((1.0 - theta) * _NPOINT)
        perm = jax.random.permutation(k_perm, _NPOINT)
        return np.asarray(perm[:new_npoint], dtype=np.int32)


_IDX_NP = _make_idx()


@functools.lru_cache(maxsize=None)
def _build():
    idx_np = _IDX_NP.astype(np.int64)
    m = int(idx_np.shape[0])
    mp = ((m + 127) // 128) * 128

    blocks, starts, counts = [], [], []
    off = 0
    for w in range(_NW):
        sel = np.nonzero((idx_np >= w * _W) & (idx_np < (w + 1) * _W))[0]
        lidx = idx_np[sel] - w * _W
        ent = ((sel << 16) | lidx).astype(np.int32)
        npad = ((len(ent) + 127) // 128) * 128
        if npad == 0:
            starts.append(off)
            counts.append(0)
            continue
        pad = np.full((npad,), ent[0] if len(ent) else 0, np.int32)
        pad[: len(ent)] = ent
        blocks.append(pad)
        starts.append(off)
        counts.append(npad)
        off += npad
    entries = np.concatenate(blocks) if blocks else np.zeros((128,), np.int32)
    ent_max = max(max(counts), 128)

    mesh = plsc.VectorSubcoreMesh(core_axis_name="c", subcore_axis_name="s")

    @functools.partial(
        pl.kernel,
        out_type=jax.ShapeDtypeStruct((_ROWS, mp), jnp.float32),
        mesh=mesh,
        compiler_params=pltpu.CompilerParams(needs_layout_passes=False),
        scratch_types=[
            pltpu.VMEM((_CH, _W), jnp.float32),
            pltpu.VMEM((ent_max,), jnp.int32),
            pltpu.VMEM((mp,), jnp.float32),
            pltpu.VMEM((mp,), jnp.float32),
            pltpu.VMEM((mp,), jnp.float32),
        ],
    )
    def _points_gather(x_hbm, ent_hbm, out_hbm, win_v, ent_v,
                       out0_v, out1_v, out2_v):
        b = lax.axis_index("s") * 2 + lax.axis_index("c")
        outs = (out0_v, out1_v, out2_v)

        for w in range(_NW):
            n = counts[w]
            if n == 0:
                continue
            pltpu.sync_copy(x_hbm.at[b, :, pl.ds(w * _W, _W)], win_v)
            pltpu.sync_copy(ent_hbm.at[pl.ds(starts[w], n)],
                            ent_v.at[pl.ds(0, n)])

            @plsc.parallel_loop(0, n, step=_LANES, unroll=4)
            def _gather_step(i):
                e = ent_v[pl.ds(i, _LANES)]
                pos = jnp.right_shift(e, 16)
                lidx = jnp.bitwise_and(e, 0xFFFF)
                for c in range(_CH):
                    cv = jnp.full((_LANES,), c, jnp.int32)
                    vals = plsc.load_gather(win_v, [cv, lidx])
                    plsc.store_scatter(outs[c], [pos], vals)

        for c in range(_CH):
            pltpu.sync_copy(outs[c], out_hbm.at[b * _CH + c])

    return _points_gather, entries, m


def kernel(xyz):
    points_gather, entries, m = _build()
    ent = jnp.asarray(entries)
    out = points_gather(xyz, ent)
    return out[:, :m].reshape(_BATCH, _CH, m)


# speedup vs baseline: 1.6656x; 1.0367x over previous
import functools

import numpy as np
import jax
import jax.numpy as jnp
from jax import lax
from jax.experimental import pallas as pl
from jax.experimental.pallas import tpu as pltpu
from jax.experimental.pallas import tpu_sc as plsc

_BATCH = 32
_CH = 3
_NPOINT = 65536
_ROWS = _BATCH * _CH
_LANES = 16
_W = 2048
_NW = _NPOINT // _W


@functools.lru_cache(maxsize=None)
def _make_idx() -> np.ndarray:
    try:
        cpu = jax.devices("cpu")[0]
        ctx = jax.default_device(cpu)
    except Exception:
        import contextlib
        ctx = contextlib.nullcontext()
    with ctx:
        key = jax.random.key(42)
        k_theta, k_perm = jax.random.split(key)
        theta = float(jax.random.uniform(k_theta, (), minval=0.0, maxval=0.95))
        new_npoint = int((1.0 - theta) * _NPOINT)
        perm = jax.random.permutation(k_perm, _NPOINT)
        return np.asarray(perm[:new_npoint], dtype=np.int32)


_IDX_NP = _make_idx()


@functools.lru_cache(maxsize=None)
def _build():
    idx_np = _IDX_NP.astype(np.int64)
    m = int(idx_np.shape[0])
    mp = ((m + 127) // 128) * 128

    blocks, starts, counts = [], [], []
    off = 0
    for w in range(_NW):
        sel = np.nonzero((idx_np >= w * _W) & (idx_np < (w + 1) * _W))[0]
        lidx = idx_np[sel] - w * _W
        ent = ((sel << 16) | lidx).astype(np.int32)
        npad = ((len(ent) + 127) // 128) * 128
        if npad == 0:
            starts.append(off)
            counts.append(0)
            continue
        pad = np.full((npad,), ent[0] if len(ent) else 0, np.int32)
        pad[: len(ent)] = ent
        blocks.append(pad)
        starts.append(off)
        counts.append(npad)
        off += npad
    entries = np.concatenate(blocks) if blocks else np.zeros((128,), np.int32)
    ent_max = max(max(counts), 128)

    mesh = plsc.VectorSubcoreMesh(core_axis_name="c", subcore_axis_name="s")

    @functools.partial(
        pl.kernel,
        out_type=jax.ShapeDtypeStruct((_ROWS, mp), jnp.float32),
        mesh=mesh,
        compiler_params=pltpu.CompilerParams(needs_layout_passes=False),
        scratch_types=[
            pltpu.VMEM((_CH, _W), jnp.float32),
            pltpu.VMEM((_CH, _W), jnp.float32),
            pltpu.VMEM((ent_max,), jnp.int32),
            pltpu.VMEM((ent_max,), jnp.int32),
            pltpu.VMEM((mp,), jnp.float32),
            pltpu.VMEM((mp,), jnp.float32),
            pltpu.VMEM((mp,), jnp.float32),
            pltpu.SemaphoreType.DMA,
            pltpu.SemaphoreType.DMA,
            pltpu.SemaphoreType.DMA,
            pltpu.SemaphoreType.DMA,
        ],
    )
    def _points_gather(x_hbm, ent_hbm, out_hbm, win0_v, win1_v,
                       ent0_v, ent1_v, out0_v, out1_v, out2_v,
                       wsem0, wsem1, esem0, esem1):
        b = lax.axis_index("s") * 2 + lax.axis_index("c")
        outs = (out0_v, out1_v, out2_v)
        wins = (win0_v, win1_v)
        ents = (ent0_v, ent1_v)
        wsems = (wsem0, wsem1)
        esems = (esem0, esem1)

        ws = [w for w in range(_NW) if counts[w] > 0]

        def start(k, slot):
            w = ws[k]
            wcp = pltpu.async_copy(
                x_hbm.at[b, :, pl.ds(w * _W, _W)], wins[slot], wsems[slot])
            ecp = pltpu.async_copy(
                ent_hbm.at[pl.ds(starts[w], counts[w])],
                ents[slot].at[pl.ds(0, counts[w])], esems[slot])
            return wcp, ecp

        inflight = {0: start(0, 0)}
        for k, w in enumerate(ws):
            slot = k % 2
            wcp, ecp = inflight.pop(slot)
            wcp.wait()
            ecp.wait()
            if k + 1 < len(ws):
                inflight[1 - slot] = start(k + 1, 1 - slot)
            n = counts[w]
            win_v = wins[slot]
            ent_v = ents[slot]

            @plsc.parallel_loop(0, n, step=_LANES, unroll=4)
            def _gather_step(i):
                e = ent_v[pl.ds(i, _LANES)]
                pos = jnp.right_shift(e, 16)
                lidx = jnp.bitwise_and(e, 0xFFFF)
                for c in range(_CH):
                    cv = jnp.full((_LANES,), c, jnp.int32)
                    vals = plsc.load_gather(win_v, [cv, lidx])
                    plsc.store_scatter(outs[c], [pos], vals)

        for c in range(_CH):
            pltpu.sync_copy(outs[c], out_hbm.at[b * _CH + c])

    return _points_gather, entries, m


def kernel(xyz):
    points_gather, entries, m = _build()
    ent = jnp.asarray(entries)
    out = points_gather(xyz, ent)
    return out[:, :m].reshape(_BATCH, _CH, m)
